# baseline (speedup 1.0000x reference)
import functools
import jax
import jax.numpy as jnp
from jax.experimental import pallas as pl


def _pad_to(a, m, axis):
    r = (-a.shape[axis]) % m
    if r == 0:
        return a
    pw = [(0, 0)] * a.ndim
    pw[axis] = (0, r)
    return jnp.pad(a, pw)


def _mm_kernel(x_ref, w_ref, b_ref, o_ref, *, relu):
    acc = jnp.dot(x_ref[...], w_ref[...], preferred_element_type=jnp.float32)
    acc = acc + b_ref[0:1, :]
    if relu:
        acc = jnp.maximum(acc, 0.0)
    o_ref[...] = acc


def _mm(x, w, b, relu=False):
    M0, K0 = x.shape
    N0 = w.shape[1]
    npad = -(-N0 // 128) * 128
    bn = min(512, npad)
    bm = 512 if M0 > 512 else -(-M0 // 8) * 8
    x = _pad_to(_pad_to(x, bm, 0), 128, 1)
    w = _pad_to(_pad_to(w, 128, 0), bn, 1)
    b8 = jnp.tile(_pad_to(b.reshape(1, -1), bn, 1), (8, 1))
    M, K = x.shape
    N = w.shape[1]
    grid = (M // bm, N // bn)
    out = pl.pallas_call(
        functools.partial(_mm_kernel, relu=relu),
        grid=grid,
        in_specs=[
            pl.BlockSpec((bm, K), lambda i, j: (i, 0)),
            pl.BlockSpec((K, bn), lambda i, j: (0, j)),
            pl.BlockSpec((8, bn), lambda i, j: (0, j)),
        ],
        out_specs=pl.BlockSpec((bm, bn), lambda i, j: (i, j)),
        out_shape=jax.ShapeDtypeStruct((M, N), jnp.float32),
    )(x, w, b8)
    return out[:M0, :N0]


def _gcn(h_in, W, b, s_all, t_all, norm, n):
    h = _mm(h_in, W, b * 0.0)
    msg = h[s_all] * norm[:, None]
    return jax.ops.segment_sum(msg, t_all, num_segments=n) + b


def kernel(x, edge_index, batch, target, W1, b1, W2, b2, W3, b3, W4, b4, W5, b5, W6, b6,
           fc_g1_w, fc_g1_b, fc_g2_w, fc_g2_b, fc_g3_w, fc_g3_b, fc_g4_w, fc_g4_b,
           emb, conv_w, conv_b, fc1_xt_w, fc1_xt_b, fc1_w, fc1_b, fc2_w, fc2_b, out_w, out_b):
    n = x.shape[0]
    B = target.shape[0]
    src = edge_index[0]
    dst = edge_index[1]
    sl = jnp.arange(n, dtype=src.dtype)
    s_all = jnp.concatenate([src, sl])
    t_all = jnp.concatenate([dst, sl])
    deg = jax.ops.segment_sum(jnp.ones_like(t_all, dtype=jnp.float32), t_all, num_segments=n)
    dinv = jnp.where(deg > 0, jax.lax.rsqrt(deg), 0.0)
    norm = dinv[s_all] * dinv[t_all]

    h = _gcn(x, W1, b1, s_all, t_all, norm, n)
    h1 = jnp.maximum(h, 0.0)
    h = _gcn(jnp.concatenate([h, h1], axis=1), W2, b2, s_all, t_all, norm, n)
    h = jnp.maximum(h, 0.0)
    h = jnp.maximum(_gcn(h, W3, b3, s_all, t_all, norm, n), 0.0)
    h = jnp.maximum(_gcn(h, W4, b4, s_all, t_all, norm, n), 0.0)
    h = jnp.maximum(_gcn(h, W5, b5, s_all, t_all, norm, n), 0.0)
    h = jnp.maximum(_gcn(h, W6, b6, s_all, t_all, norm, n), 0.0)

    g = jax.ops.segment_max(h, batch, num_segments=B)
    g = _mm(g, fc_g1_w, fc_g1_b, relu=True)
    g = _mm(g, fc_g2_w, fc_g2_b, relu=True)
    g = _mm(g, fc_g3_w, fc_g3_b, relu=True)
    g = _mm(g, fc_g4_w, fc_g4_b)

    e = emb[target]
    P = e.shape[2] - conv_w.shape[2] + 1
    wins = jnp.stack([e[:, :, p:p + conv_w.shape[2]] for p in range(P)], axis=1)
    x2 = wins.reshape(B * P, conv_w.shape[1] * conv_w.shape[2])
    w2 = conv_w.reshape(conv_w.shape[0], -1).T
    c = _mm(x2, w2, conv_b)
    c = c.reshape(B, P, conv_w.shape[0]).transpose(0, 2, 1)
    xt = _mm(c.reshape(B, -1), fc1_xt_w, fc1_xt_b)

    xc = jnp.concatenate([g, xt], axis=1)
    xc = _mm(xc, fc1_w, fc1_b, relu=True)
    xc = _mm(xc, fc2_w, fc2_b, relu=True)
    return _mm(xc, out_w, out_b)


# speedup vs baseline: 1.2281x; 1.2281x over previous
import functools
import jax
import jax.numpy as jnp
from jax.experimental import pallas as pl


def _row_block(m):
    for bm in (512, 400, 352, 256, 128, 64, 32, 16, 8):
        if m % bm == 0:
            return bm
    return m


def _mm_kernel(x_ref, w_ref, b_ref, o_ref, *, relu):
    acc = jnp.dot(x_ref[...], w_ref[...], preferred_element_type=jnp.float32)
    acc = acc + b_ref[0:1, :]
    if relu:
        acc = jnp.maximum(acc, 0.0)
    o_ref[...] = acc


def _mm(x, w, b, relu=False):
    M, K = x.shape
    N = w.shape[1]
    bm = _row_block(M)
    b8 = jnp.tile(b.reshape(1, -1), (8, 1))
    grid = (M // bm,)
    return pl.pallas_call(
        functools.partial(_mm_kernel, relu=relu),
        grid=grid,
        in_specs=[
            pl.BlockSpec((bm, K), lambda i: (i, 0)),
            pl.BlockSpec((K, N), lambda i: (0, 0)),
            pl.BlockSpec((8, N), lambda i: (0, 0)),
        ],
        out_specs=pl.BlockSpec((bm, N), lambda i: (i, 0)),
        out_shape=jax.ShapeDtypeStruct((M, N), jnp.float32),
    )(x, w, b8)


def _gcn(h_in, W, b, s_all, t_all, dinv, n):
    h = _mm(h_in, W, jnp.zeros_like(b))
    hs = h * dinv[:, None]
    agg = jax.ops.segment_sum(hs[s_all], t_all, num_segments=n)
    return agg * dinv[:, None] + b


def kernel(x, edge_index, batch, target, W1, b1, W2, b2, W3, b3, W4, b4, W5, b5, W6, b6,
           fc_g1_w, fc_g1_b, fc_g2_w, fc_g2_b, fc_g3_w, fc_g3_b, fc_g4_w, fc_g4_b,
           emb, conv_w, conv_b, fc1_xt_w, fc1_xt_b, fc1_w, fc1_b, fc2_w, fc2_b, out_w, out_b):
    n = x.shape[0]
    B = target.shape[0]
    src = edge_index[0]
    dst = edge_index[1]
    sl = jnp.arange(n, dtype=src.dtype)
    s_all = jnp.concatenate([src, sl])
    t_all = jnp.concatenate([dst, sl])
    deg = jax.ops.segment_sum(jnp.ones_like(t_all, dtype=jnp.float32), t_all, num_segments=n)
    dinv = jnp.where(deg > 0, jax.lax.rsqrt(deg), 0.0)

    h = _gcn(x, W1, b1, s_all, t_all, dinv, n)
    h1 = jnp.maximum(h, 0.0)
    h = _gcn(jnp.concatenate([h, h1], axis=1), W2, b2, s_all, t_all, dinv, n)
    h = jnp.maximum(h, 0.0)
    h = jnp.maximum(_gcn(h, W3, b3, s_all, t_all, dinv, n), 0.0)
    h = jnp.maximum(_gcn(h, W4, b4, s_all, t_all, dinv, n), 0.0)
    h = jnp.maximum(_gcn(h, W5, b5, s_all, t_all, dinv, n), 0.0)
    h = jnp.maximum(_gcn(h, W6, b6, s_all, t_all, dinv, n), 0.0)

    g = jax.ops.segment_max(h, batch, num_segments=B)
    g = _mm(g, fc_g1_w, fc_g1_b, relu=True)
    g = _mm(g, fc_g2_w, fc_g2_b, relu=True)
    g = _mm(g, fc_g3_w, fc_g3_b, relu=True)
    g = _mm(g, fc_g4_w, fc_g4_b)

    e = emb[target]
    P = e.shape[2] - conv_w.shape[2] + 1
    wins = jnp.stack([e[:, :, p:p + conv_w.shape[2]] for p in range(P)], axis=1)
    x2 = wins.reshape(B * P, conv_w.shape[1] * conv_w.shape[2])
    w2 = conv_w.reshape(conv_w.shape[0], -1).T
    c = _mm(x2, w2, conv_b)
    c = c.reshape(B, P, conv_w.shape[0]).transpose(0, 2, 1)
    xt = _mm(c.reshape(B, -1), fc1_xt_w, fc1_xt_b)

    xc = jnp.concatenate([g, xt], axis=1)
    xc = _mm(xc, fc1_w, fc1_b, relu=True)
    xc = _mm(xc, fc2_w, fc2_b, relu=True)
    return _mm(xc, out_w, out_b)
